# baseline (device time: 796963 ns/iter reference)
import jax
import jax.numpy as jnp
from jax import lax
from jax.experimental import pallas as pl
from jax.experimental.pallas import tpu as pltpu

N_DEV = 32


def kernel(x, w_mat):
    m, k_loc = x.shape
    k2, n = w_mat.shape
    chunk = m // N_DEV

    x = x.astype(jnp.bfloat16)
    w = w_mat.astype(jnp.bfloat16)

    def body(x_ref, w_ref, out_ref, send_buf, recv_buf,
             send_sems, recv_sems, credit_sem):
        d = lax.axis_index("i")
        left = lax.rem(d + (N_DEV - 1), N_DEV)
        right = lax.rem(d + 1, N_DEV)

        barrier_sem = pltpu.get_barrier_semaphore()
        for nbr in (left, right):
            pl.semaphore_signal(barrier_sem, inc=1, device_id=(nbr,),
                                device_id_type=pl.DeviceIdType.MESH)
        pl.semaphore_wait(barrier_sem, 2)

        def partial(c):
            xs = x_ref[pl.ds(c * chunk, chunk), :]
            return jnp.dot(xs, w_ref[:, :],
                           preferred_element_type=jnp.float32)

        send_buf[0] = partial(lax.rem(d + (N_DEV - 1), N_DEV)).astype(
            jnp.bfloat16)

        acc = None
        for h in range(N_DEV - 1):
            s = h % 2
            if h >= 2:
                pl.semaphore_wait(credit_sem, 1)
            rdma = pltpu.make_async_remote_copy(
                src_ref=send_buf.at[s],
                dst_ref=recv_buf.at[s],
                send_sem=send_sems.at[s],
                recv_sem=recv_sems.at[s],
                device_id=(right,),
                device_id_type=pl.DeviceIdType.MESH,
            )
            rdma.start()
            rc = lax.rem(d + (2 * N_DEV - h - 2), N_DEV)
            p = partial(rc)
            rdma.wait()
            acc = p + recv_buf[s].astype(jnp.float32)
            pl.semaphore_signal(credit_sem, inc=1, device_id=(left,),
                                device_id_type=pl.DeviceIdType.MESH)
            if h < N_DEV - 2:
                send_buf[(h + 1) % 2] = acc.astype(jnp.bfloat16)

        out_ref[:, :] = acc * jax.nn.sigmoid(acc)

        pl.semaphore_wait(credit_sem, 2)

    return pl.pallas_call(
        body,
        out_shape=jax.ShapeDtypeStruct((chunk, n), jnp.float32),
        in_specs=[
            pl.BlockSpec(memory_space=pltpu.VMEM),
            pl.BlockSpec(memory_space=pltpu.VMEM),
        ],
        out_specs=pl.BlockSpec(memory_space=pltpu.VMEM),
        scratch_shapes=[
            pltpu.VMEM((2, chunk, n), jnp.bfloat16),
            pltpu.VMEM((2, chunk, n), jnp.bfloat16),
            pltpu.SemaphoreType.DMA((2,)),
            pltpu.SemaphoreType.DMA((2,)),
            pltpu.SemaphoreType.REGULAR,
        ],
        compiler_params=pltpu.CompilerParams(collective_id=0),
    )(x, w)


# device time: 449254 ns/iter; 1.7740x vs baseline; 1.7740x over previous
import jax
import jax.numpy as jnp
import numpy as np
from jax import lax
from jax.experimental import pallas as pl
from jax.experimental.pallas import tpu as pltpu

N_DEV = 32


def _ring_order():
    try:
        devs = [d for d in jax.devices()
                if getattr(d, "core_on_chip", 1) == 1]
        coords = sorted(tuple(d.coords) for d in devs)
    except Exception:
        return list(range(N_DEV))
    if len(coords) != N_DEV:
        return list(range(N_DEV))
    xs = sorted({c[0] for c in coords})
    ys = sorted({c[1] for c in coords})
    zs = sorted({c[2] for c in coords})
    if (len(xs), len(ys), len(zs)) != (2, 4, 4):
        return list(range(N_DEV))
    logical = {}
    for zi, z in enumerate(zs):
        for yi, y in enumerate(ys):
            row = xs if yi % 2 == 0 else xs[::-1]
            for k, x in enumerate(row):
                logical[(x, y, z)] = 8 * zi + 2 * yi + k
    snake = []
    for zi, z in enumerate(zs):
        yy = ys if zi % 2 == 0 else ys[::-1]
        snake.extend((y, z) for y in yy)
    cycle = ([(xs[0], y, z) for (y, z) in snake]
             + [(xs[1], y, z) for (y, z) in reversed(snake)])
    for a, b in zip(cycle, cycle[1:] + cycle[:1]):
        if sum(abs(u - v) for u, v in zip(a, b)) != 1:
            return list(range(N_DEV))
    return [logical[c] for c in cycle]


def kernel(x, w_mat):
    m, k_loc = x.shape
    k2, n = w_mat.shape
    chunk = m // N_DEV
    half = n // 2

    x = x.astype(jnp.bfloat16)
    w = w_mat.astype(jnp.bfloat16)

    ring = _ring_order()
    ring_arr = jnp.asarray(np.array(ring, dtype=np.int32))
    pos = np.empty(N_DEV, dtype=np.int32)
    for i, dev in enumerate(ring):
        pos[dev] = i
    pos_arr = jnp.asarray(pos)

    d = lax.axis_index("i")
    r = pos_arr[d]
    right = ring_arr[(r + 1) % N_DEV]
    left = ring_arr[(r - 1) % N_DEV]
    hops = jnp.arange(N_DEV, dtype=jnp.int32)
    cw_idx = ring_arr[(r - 1 - hops) % N_DEV]
    ccw_idx = ring_arr[(r + 1 + hops) % N_DEV]
    meta = jnp.concatenate(
        [right[None], left[None], cw_idx, ccw_idx]).astype(jnp.int32)

    def body(meta_ref, x_ref, w_ref, out_ref,
             send_cw, recv_cw, send_ccw, recv_ccw,
             sems_cw, sems_ccw, credit_cw, credit_ccw):
        rt = meta_ref[0]
        lt = meta_ref[1]

        barrier_sem = pltpu.get_barrier_semaphore()
        for nbr in (lt, rt):
            pl.semaphore_signal(barrier_sem, inc=1, device_id=(nbr,),
                                device_id_type=pl.DeviceIdType.MESH)
        pl.semaphore_wait(barrier_sem, 2)

        def partial(c, col0):
            xs_ = x_ref[pl.ds(c * chunk, chunk), :]
            return jnp.dot(xs_, w_ref[:, pl.ds(col0, half)],
                           preferred_element_type=jnp.float32)

        send_cw[0] = partial(meta_ref[2], 0).astype(jnp.bfloat16)
        send_ccw[0] = partial(meta_ref[2 + N_DEV], half).astype(jnp.bfloat16)

        acc_cw = acc_ccw = None
        for h in range(N_DEV - 1):
            s = h % 2
            if h >= 2:
                pl.semaphore_wait(credit_cw, 1)
                pl.semaphore_wait(credit_ccw, 1)
            rdma_cw = pltpu.make_async_remote_copy(
                src_ref=send_cw.at[s], dst_ref=recv_cw.at[s],
                send_sem=sems_cw.at[0, s], recv_sem=sems_cw.at[1, s],
                device_id=(rt,), device_id_type=pl.DeviceIdType.MESH,
            )
            rdma_ccw = pltpu.make_async_remote_copy(
                src_ref=send_ccw.at[s], dst_ref=recv_ccw.at[s],
                send_sem=sems_ccw.at[0, s], recv_sem=sems_ccw.at[1, s],
                device_id=(lt,), device_id_type=pl.DeviceIdType.MESH,
            )
            rdma_cw.start()
            rdma_ccw.start()
            p_cw = partial(meta_ref[2 + h + 1], 0)
            p_ccw = partial(meta_ref[2 + N_DEV + h + 1], half)
            rdma_cw.wait()
            rdma_ccw.wait()
            acc_cw = p_cw + recv_cw[s].astype(jnp.float32)
            acc_ccw = p_ccw + recv_ccw[s].astype(jnp.float32)
            pl.semaphore_signal(credit_cw, inc=1, device_id=(lt,),
                                device_id_type=pl.DeviceIdType.MESH)
            pl.semaphore_signal(credit_ccw, inc=1, device_id=(rt,),
                                device_id_type=pl.DeviceIdType.MESH)
            if h < N_DEV - 2:
                send_cw[(h + 1) % 2] = acc_cw.astype(jnp.bfloat16)
                send_ccw[(h + 1) % 2] = acc_ccw.astype(jnp.bfloat16)

        out_ref[:, :half] = acc_cw * jax.nn.sigmoid(acc_cw)
        out_ref[:, half:] = acc_ccw * jax.nn.sigmoid(acc_ccw)

        pl.semaphore_wait(credit_cw, 2)
        pl.semaphore_wait(credit_ccw, 2)

    return pl.pallas_call(
        body,
        out_shape=jax.ShapeDtypeStruct((chunk, n), jnp.float32),
        in_specs=[
            pl.BlockSpec(memory_space=pltpu.SMEM),
            pl.BlockSpec(memory_space=pltpu.VMEM),
            pl.BlockSpec(memory_space=pltpu.VMEM),
        ],
        out_specs=pl.BlockSpec(memory_space=pltpu.VMEM),
        scratch_shapes=[
            pltpu.VMEM((2, chunk, half), jnp.bfloat16),
            pltpu.VMEM((2, chunk, half), jnp.bfloat16),
            pltpu.VMEM((2, chunk, half), jnp.bfloat16),
            pltpu.VMEM((2, chunk, half), jnp.bfloat16),
            pltpu.SemaphoreType.DMA((2, 2)),
            pltpu.SemaphoreType.DMA((2, 2)),
            pltpu.SemaphoreType.REGULAR,
            pltpu.SemaphoreType.REGULAR,
        ],
        compiler_params=pltpu.CompilerParams(collective_id=0),
    )(meta, x, w)


# device time: 369774 ns/iter; 2.1553x vs baseline; 1.2149x over previous
import jax
import jax.numpy as jnp
import numpy as np
from jax import lax
from jax.experimental import pallas as pl
from jax.experimental.pallas import tpu as pltpu

N_DEV = 32
N_SUB = 2


def _ring_order():
    try:
        devs = [d for d in jax.devices()
                if getattr(d, "core_on_chip", 1) == 1]
        coords = sorted(tuple(d.coords) for d in devs)
    except Exception:
        return list(range(N_DEV))
    if len(coords) != N_DEV:
        return list(range(N_DEV))
    xs = sorted({c[0] for c in coords})
    ys = sorted({c[1] for c in coords})
    zs = sorted({c[2] for c in coords})
    if (len(xs), len(ys), len(zs)) != (2, 4, 4):
        return list(range(N_DEV))
    logical = {}
    for zi, z in enumerate(zs):
        for yi, y in enumerate(ys):
            row = xs if yi % 2 == 0 else xs[::-1]
            for k, x in enumerate(row):
                logical[(x, y, z)] = 8 * zi + 2 * yi + k
    snake = []
    for zi, z in enumerate(zs):
        yy = ys if zi % 2 == 0 else ys[::-1]
        snake.extend((y, z) for y in yy)
    cycle = ([(xs[0], y, z) for (y, z) in snake]
             + [(xs[1], y, z) for (y, z) in reversed(snake)])
    for a, b in zip(cycle, cycle[1:] + cycle[:1]):
        if sum(abs(u - v) for u, v in zip(a, b)) != 1:
            return list(range(N_DEV))
    return [logical[c] for c in cycle]


def kernel(x, w_mat):
    m, k_loc = x.shape
    k2, n = w_mat.shape
    chunk = m // N_DEV
    half = n // 2
    sub = half // N_SUB

    x = x.astype(jnp.bfloat16)
    w = w_mat.astype(jnp.bfloat16)

    ring = _ring_order()
    ring_arr = jnp.asarray(np.array(ring, dtype=np.int32))
    pos = np.empty(N_DEV, dtype=np.int32)
    for i, dev in enumerate(ring):
        pos[dev] = i
    pos_arr = jnp.asarray(pos)

    d = lax.axis_index("i")
    r = pos_arr[d]
    right = ring_arr[(r + 1) % N_DEV]
    left = ring_arr[(r - 1) % N_DEV]
    hops = jnp.arange(N_DEV, dtype=jnp.int32)
    cw_idx = ring_arr[(r - 1 - hops) % N_DEV]
    ccw_idx = ring_arr[(r + 1 + hops) % N_DEV]
    meta = jnp.concatenate(
        [right[None], left[None], cw_idx, ccw_idx]).astype(jnp.int32)

    def body(meta_ref, x_ref, w_ref, out_ref,
             send_cw, recv_cw, send_ccw, recv_ccw,
             ssem_cw, rsem_cw, ssem_ccw, rsem_ccw,
             cred_cw0, cred_cw1, cred_ccw0, cred_ccw1):
        rt = meta_ref[0]
        lt = meta_ref[1]
        creds = {("cw", 0): cred_cw0, ("cw", 1): cred_cw1,
                 ("ccw", 0): cred_ccw0, ("ccw", 1): cred_ccw1}
        bufs = {"cw": (send_cw, recv_cw, ssem_cw, rsem_cw, rt, lt, 0),
                "ccw": (send_ccw, recv_ccw, ssem_ccw, rsem_ccw, lt, rt,
                        half)}

        barrier_sem = pltpu.get_barrier_semaphore()
        for nbr in (lt, rt):
            pl.semaphore_signal(barrier_sem, inc=1, device_id=(nbr,),
                                device_id_type=pl.DeviceIdType.MESH)
        pl.semaphore_wait(barrier_sem, 2)

        def partial(c, col0):
            xs_ = x_ref[pl.ds(c * chunk, chunk), :]
            return jnp.dot(xs_, w_ref[:, pl.ds(col0, sub)],
                           preferred_element_type=jnp.float32)

        def chunk_at(dirn, h):
            off = 2 if dirn == "cw" else 2 + N_DEV
            return meta_ref[off + h]

        def mk(dirn, h, j):
            sbuf, rbuf, ssem, rsem, dst, _, _ = bufs[dirn]
            sl = h % 2
            return pltpu.make_async_remote_copy(
                src_ref=sbuf.at[sl, j], dst_ref=rbuf.at[sl, j],
                send_sem=ssem.at[sl, j], recv_sem=rsem.at[sl, j],
                device_id=(dst,), device_id_type=pl.DeviceIdType.MESH,
            )

        sent = {}

        for dirn in ("cw", "ccw"):
            sbuf = bufs[dirn][0]
            col0 = bufs[dirn][6]
            for j in range(N_SUB):
                sbuf[0, j] = partial(chunk_at(dirn, 0),
                                     col0 + j * sub).astype(jnp.bfloat16)
                rdma = mk(dirn, 0, j)
                rdma.start()
                sent[(dirn, 0, j)] = rdma

        for h in range(1, N_DEV - 1):
            sl_prev = (h - 1) % 2
            sl = h % 2
            for j in range(N_SUB):
                for dirn in ("cw", "ccw"):
                    sbuf, rbuf, ssem, rsem, dst, src, col0 = bufs[dirn]
                    mk(dirn, h - 1, j).wait_recv()
                    val = rbuf[sl_prev, j].astype(jnp.float32) \
                        + partial(chunk_at(dirn, h), col0 + j * sub)
                    pl.semaphore_signal(creds[(dirn, j)], inc=1,
                                        device_id=(src,),
                                        device_id_type=pl.DeviceIdType.MESH)
                    if h >= 2:
                        sent[(dirn, h - 2, j)].wait_send()
                    sbuf[sl, j] = val.astype(jnp.bfloat16)
                    if h >= 2:
                        pl.semaphore_wait(creds[(dirn, j)], 1)
                    rdma = mk(dirn, h, j)
                    rdma.start()
                    sent[(dirn, h, j)] = rdma

        for j in range(N_SUB):
            for dirn in ("cw", "ccw"):
                _, rbuf, _, _, _, src, col0 = bufs[dirn]
                mk(dirn, N_DEV - 2, j).wait_recv()
                acc = rbuf[0, j].astype(jnp.float32) \
                    + partial(chunk_at(dirn, N_DEV - 1), col0 + j * sub)
                out_ref[:, pl.ds(col0 + j * sub, sub)] = \
                    acc * jax.nn.sigmoid(acc)
                pl.semaphore_signal(creds[(dirn, j)], inc=1,
                                    device_id=(src,),
                                    device_id_type=pl.DeviceIdType.MESH)

        for dirn in ("cw", "ccw"):
            for j in range(N_SUB):
                pl.semaphore_wait(creds[(dirn, j)], 2)
                sent[(dirn, N_DEV - 3, j)].wait_send()
                sent[(dirn, N_DEV - 2, j)].wait_send()

    return pl.pallas_call(
        body,
        out_shape=jax.ShapeDtypeStruct((chunk, n), jnp.float32),
        in_specs=[
            pl.BlockSpec(memory_space=pltpu.SMEM),
            pl.BlockSpec(memory_space=pltpu.VMEM),
            pl.BlockSpec(memory_space=pltpu.VMEM),
        ],
        out_specs=pl.BlockSpec(memory_space=pltpu.VMEM),
        scratch_shapes=[
            pltpu.VMEM((2, N_SUB, chunk, sub), jnp.bfloat16),
            pltpu.VMEM((2, N_SUB, chunk, sub), jnp.bfloat16),
            pltpu.VMEM((2, N_SUB, chunk, sub), jnp.bfloat16),
            pltpu.VMEM((2, N_SUB, chunk, sub), jnp.bfloat16),
            pltpu.SemaphoreType.DMA((2, N_SUB)),
            pltpu.SemaphoreType.DMA((2, N_SUB)),
            pltpu.SemaphoreType.DMA((2, N_SUB)),
            pltpu.SemaphoreType.DMA((2, N_SUB)),
            pltpu.SemaphoreType.REGULAR,
            pltpu.SemaphoreType.REGULAR,
            pltpu.SemaphoreType.REGULAR,
            pltpu.SemaphoreType.REGULAR,
        ],
        compiler_params=pltpu.CompilerParams(collective_id=0),
    )(meta, x, w)
